# Initial kernel scaffold; baseline (speedup 1.0000x reference)
import functools

import jax
import jax.numpy as jnp
from jax import lax
from jax.experimental import pallas as pl
from jax.experimental.pallas import tpu as pltpu
from jax.experimental.pallas import tpu_sc as plsc

_N = 10000
_NPAD = 10240
_K = 128
_NW = 32
_BLK = 1024
_F32 = jnp.float32


def _splat(vec, lane):
    idx = jnp.full((16,), lane, dtype=jnp.int32)
    return jnp.take(vec, idx, axis=0, mode="promise_in_bounds")


def _make_edge_pass(head_of_slice, cpw, name):
    mesh = plsc.VectorSubcoreMesh(core_axis_name="c", subcore_axis_name="s")
    rows_per_sub = _NPAD // 16

    @functools.partial(
        pl.kernel,
        out_type=(jax.ShapeDtypeStruct((2, _NPAD, 128), _F32),
                  jax.ShapeDtypeStruct((2, _NPAD, 16), _F32)),
        mesh=mesh,
        scratch_types=[
            pltpu.VMEM((_K,), jnp.int32),
            pltpu.VMEM((_K,), jnp.int32),
            pltpu.VMEM((_K, 144), _F32),
            pltpu.VMEM((_K, 16), _F32),
            pltpu.VMEM((_K, 128), _F32),
            pltpu.VMEM((_K, 16), _F32),
            pltpu.VMEM_SHARED((_NPAD, 128), _F32),
            pltpu.VMEM_SHARED((_NPAD, 16), _F32),
            pltpu.SemaphoreType.DMA,
            pltpu.SemaphoreType.DMA,
        ],
        name=name,
    )
    def edge_pass(hp, adst, src, dst, outp, outd,
                  src_v, dst_v, hp_b, adst_b, msg_b, w_b, acc_s, den_s,
                  sem1, sem2):
        core = lax.axis_index("c")
        sub = lax.axis_index("s")
        wid = sub * 2 + core
        zero16 = jnp.zeros((16,), _F32)

        @pl.loop(0, _K)
        def _(j):
            w_b[j] = zero16
            for c in range(8):
                msg_b[j, pl.ds(c * 16, 16)] = zero16

        @pl.loop(0, rows_per_sub // _K)
        def _(t):
            off = sub * rows_per_sub + t * _K
            pltpu.sync_copy(msg_b, acc_s.at[pl.ds(off, _K)])
            pltpu.sync_copy(w_b, den_s.at[pl.ds(off, _K)])

        plsc.subcore_barrier()

        @pl.loop(0, cpw)
        def _(ci):
            base = (wid * cpw + ci) * _K
            pltpu.sync_copy(src.at[pl.ds(base, _K)], src_v)
            pltpu.sync_copy(dst.at[pl.ds(base, _K)], dst_v)
            cp1 = pltpu.async_copy(hp.at[src_v], hp_b, sem1)
            cp2 = pltpu.async_copy(adst.at[dst_v], adst_b, sem2)
            cp1.wait()
            cp2.wait()

            @pl.loop(0, _K)
            def _(j):
                v = hp_b[j, pl.ds(128, 16)] + adst_b[j]
                w = jnp.exp(jnp.maximum(v, 0.2 * v))
                w_b[j] = w
                splats = {}
                for c in range(8):
                    ln = head_of_slice[c]
                    if ln not in splats:
                        splats[ln] = _splat(w, ln)
                    msg_b[j, pl.ds(c * 16, 16)] = (
                        hp_b[j, pl.ds(c * 16, 16)] * splats[ln])

            pltpu.sync_copy(msg_b, acc_s.at[dst_v], add=True)
            pltpu.sync_copy(w_b, den_s.at[dst_v], add=True)

        plsc.subcore_barrier()
        off = sub * rows_per_sub
        pltpu.sync_copy(acc_s.at[pl.ds(off, rows_per_sub)],
                        outp.at[core, pl.ds(off, rows_per_sub)])
        pltpu.sync_copy(den_s.at[pl.ds(off, rows_per_sub)],
                        outd.at[core, pl.ds(off, rows_per_sub)])

    return edge_pass


def _dense1_body(x_r, W_r, as_r, ad_r, hp_r, adst_r):
    h = jnp.dot(x_r[...], W_r[...], preferred_element_type=_F32)
    B = h.shape[0]
    h3 = h.reshape(B, 8, 16)
    a_s = jnp.sum(h3 * as_r[...][None], axis=-1)
    a_d = jnp.sum(h3 * ad_r[...][None], axis=-1)
    hp_r[...] = jnp.concatenate([h, a_s, jnp.zeros((B, 8), _F32)], axis=1)
    adst_r[...] = jnp.concatenate([a_d, jnp.zeros((B, 8), _F32)], axis=1)


def _dense2_body(p_r, d_r, b1_r, W2_r, as2_r, ad2_r, hp_r, adst_r):
    p = p_r[0] + p_r[1]
    den = d_r[0] + d_r[1]
    B = p.shape[0]
    h3 = p.reshape(B, 8, 16) / (den[:, :8].reshape(B, 8, 1) + 1e-16)
    h = h3.reshape(B, 128) + b1_r[...]
    h = jnp.where(h > 0, h, jnp.expm1(h))
    g = jnp.dot(h, W2_r[...], preferred_element_type=_F32)
    a_s = jnp.sum(g * as2_r[...], axis=-1, keepdims=True)
    a_d = jnp.sum(g * ad2_r[...], axis=-1, keepdims=True)
    hp_r[...] = jnp.concatenate([g, a_s, jnp.zeros((B, 15), _F32)], axis=1)
    adst_r[...] = jnp.concatenate([a_d, jnp.zeros((B, 15), _F32)], axis=1)


def _final_body(p_r, d_r, b2_r, o_r):
    p = p_r[0] + p_r[1]
    den = d_r[0][:, :1] + d_r[1][:, :1]
    o_r[...] = p / (den + 1e-16) + b2_r[...]


def _row_spec(w):
    return pl.BlockSpec((_BLK, w), lambda i: (i, 0))


def _full_spec(shape):
    return pl.BlockSpec(shape, lambda i: tuple(0 for _ in shape))


def kernel(x, edge_index, W1, att_src1, att_dst1, b1, W2, att_src2,
           att_dst2, b2):
    E = edge_index.shape[1]
    e_all = E + _N
    tot = _NW * _K
    cpw = -(-e_all // tot)
    e_pad = cpw * tot

    loop = jnp.arange(_N, dtype=jnp.int32)
    padi = jnp.full((e_pad - e_all,), _N, dtype=jnp.int32)
    src = jnp.concatenate([edge_index[0].astype(jnp.int32), loop, padi])
    dst = jnp.concatenate([edge_index[1].astype(jnp.int32), loop, padi])
    x_pad = jnp.pad(x, ((0, _NPAD - _N), (0, 0)))

    grid = (_NPAD // _BLK,)

    hp1, adst1 = pl.pallas_call(
        _dense1_body,
        grid=grid,
        in_specs=[_row_spec(128), _full_spec((128, 128)),
                  _full_spec((8, 16)), _full_spec((8, 16))],
        out_specs=[_row_spec(144), _row_spec(16)],
        out_shape=[jax.ShapeDtypeStruct((_NPAD, 144), _F32),
                   jax.ShapeDtypeStruct((_NPAD, 16), _F32)],
    )(x_pad, W1, att_src1, att_dst1)

    ep1 = _make_edge_pass((0, 1, 2, 3, 4, 5, 6, 7), cpw, "gat_edges_l1")
    p1, d1 = ep1(hp1, adst1, src, dst)

    hp2, adst2 = pl.pallas_call(
        _dense2_body,
        grid=grid,
        in_specs=[pl.BlockSpec((2, _BLK, 128), lambda i: (0, i, 0)),
                  pl.BlockSpec((2, _BLK, 16), lambda i: (0, i, 0)),
                  _full_spec((1, 128)), _full_spec((128, 128)),
                  _full_spec((1, 128)), _full_spec((1, 128))],
        out_specs=[_row_spec(144), _row_spec(16)],
        out_shape=[jax.ShapeDtypeStruct((_NPAD, 144), _F32),
                   jax.ShapeDtypeStruct((_NPAD, 16), _F32)],
    )(p1, d1, b1.reshape(1, 128), W2, att_src2, att_dst2)

    ep2 = _make_edge_pass((0,) * 8, cpw, "gat_edges_l2")
    p2, d2 = ep2(hp2, adst2, src, dst)

    out = pl.pallas_call(
        _final_body,
        grid=grid,
        in_specs=[pl.BlockSpec((2, _BLK, 128), lambda i: (0, i, 0)),
                  pl.BlockSpec((2, _BLK, 16), lambda i: (0, i, 0)),
                  _full_spec((1, 128))],
        out_specs=_row_spec(128),
        out_shape=jax.ShapeDtypeStruct((_NPAD, 128), _F32),
    )(p2, d2, b2.reshape(1, 128))

    return out[:_N]


# speedup vs baseline: 28.1927x; 28.1927x over previous
import functools

import jax
import jax.numpy as jnp
from jax import lax
from jax.experimental import pallas as pl
from jax.experimental.pallas import tpu as pltpu
from jax.experimental.pallas import tpu_sc as plsc

_N = 10000
_NPAD = 10176
_K = 128
_NW = 32
_BLK = 848
_F32 = jnp.float32


def _splat(vec, lane):
    idx = jnp.full((16, 1), lane, dtype=jnp.int32)
    dnums = lax.GatherDimensionNumbers(
        offset_dims=(), collapsed_slice_dims=(0,), start_index_map=(0,))
    return lax.gather(vec, idx, dnums, (1,),
                      mode=lax.GatherScatterMode.PROMISE_IN_BOUNDS)


def _make_edge_pass(head_of_slice, cpw, name):
    mesh = plsc.VectorSubcoreMesh(core_axis_name="c", subcore_axis_name="s")
    rows_per_sub = _NPAD // 16

    @functools.partial(
        pl.kernel,
        out_type=(jax.ShapeDtypeStruct((2, _NPAD, 128), _F32),
                  jax.ShapeDtypeStruct((2, _NPAD, 16), _F32)),
        mesh=mesh,
        scratch_types=[
            pltpu.VMEM((_K,), jnp.int32),
            pltpu.VMEM((_K,), jnp.int32),
            pltpu.VMEM((_K, 144), _F32),
            pltpu.VMEM((_K, 16), _F32),
            pltpu.VMEM((_K, 128), _F32),
            pltpu.VMEM((_K, 16), _F32),
            pltpu.VMEM_SHARED((_NPAD, 128), _F32),
            pltpu.VMEM_SHARED((_NPAD, 16), _F32),
            pltpu.SemaphoreType.DMA,
            pltpu.SemaphoreType.DMA,
        ],
        compiler_params=pltpu.CompilerParams(use_tc_tiling_on_sc=False),
        name=name,
    )
    def edge_pass(hp, adst, src, dst, outp, outd,
                  src_v, dst_v, hp_b, adst_b, msg_b, w_b, acc_s, den_s,
                  sem1, sem2):
        core = lax.axis_index("c")
        sub = lax.axis_index("s")
        wid = sub * 2 + core
        zero16 = jnp.zeros((16,), _F32)

        @pl.loop(0, _K)
        def _(j):
            w_b[j] = zero16
            for c in range(8):
                msg_b[j, pl.ds(c * 16, 16)] = zero16

        @pl.loop(0, rows_per_sub // _K)
        def _(t):
            off = sub * rows_per_sub + t * _K
            pltpu.sync_copy(msg_b, acc_s.at[pl.ds(off, _K)])
            pltpu.sync_copy(w_b, den_s.at[pl.ds(off, _K)])

        rem = rows_per_sub % _K
        if rem:
            off0 = sub * rows_per_sub + (rows_per_sub // _K) * _K
            pltpu.sync_copy(msg_b.at[pl.ds(0, rem)], acc_s.at[pl.ds(off0, rem)])
            pltpu.sync_copy(w_b.at[pl.ds(0, rem)], den_s.at[pl.ds(off0, rem)])

        plsc.subcore_barrier()

        @pl.loop(0, cpw)
        def _(ci):
            base = (wid * cpw + ci) * _K
            pltpu.sync_copy(src.at[pl.ds(base, _K)], src_v)
            pltpu.sync_copy(dst.at[pl.ds(base, _K)], dst_v)
            cp1 = pltpu.async_copy(hp.at[src_v], hp_b, sem1)
            cp2 = pltpu.async_copy(adst.at[dst_v], adst_b, sem2)
            cp1.wait()
            cp2.wait()

            @pl.loop(0, _K)
            def _(j):
                v = hp_b[j, pl.ds(128, 16)] + adst_b[j]
                w = jnp.exp(jnp.maximum(v, 0.2 * v))
                w_b[j] = w
                splats = {}
                for c in range(8):
                    ln = head_of_slice[c]
                    if ln not in splats:
                        splats[ln] = _splat(w, ln)
                    msg_b[j, pl.ds(c * 16, 16)] = (
                        hp_b[j, pl.ds(c * 16, 16)] * splats[ln])

            pltpu.sync_copy(msg_b, acc_s.at[dst_v], add=True)
            pltpu.sync_copy(w_b, den_s.at[dst_v], add=True)

        plsc.subcore_barrier()
        off = sub * rows_per_sub
        pltpu.sync_copy(acc_s.at[pl.ds(off, rows_per_sub)],
                        outp.at[core, pl.ds(off, rows_per_sub)])
        pltpu.sync_copy(den_s.at[pl.ds(off, rows_per_sub)],
                        outd.at[core, pl.ds(off, rows_per_sub)])

    return edge_pass


def _dense1_body(x_r, W_r, as_r, ad_r, hp_r, adst_r):
    h = jnp.dot(x_r[...], W_r[...], preferred_element_type=_F32)
    B = h.shape[0]
    h3 = h.reshape(B, 8, 16)
    a_s = jnp.sum(h3 * as_r[...][None], axis=-1)
    a_d = jnp.sum(h3 * ad_r[...][None], axis=-1)
    hp_r[...] = jnp.concatenate([h, a_s, jnp.zeros((B, 8), _F32)], axis=1)
    adst_r[...] = jnp.concatenate([a_d, jnp.zeros((B, 8), _F32)], axis=1)


def _dense2_body(p_r, d_r, b1_r, W2_r, as2_r, ad2_r, hp_r, adst_r):
    p = p_r[0] + p_r[1]
    den = d_r[0] + d_r[1]
    B = p.shape[0]
    h3 = p.reshape(B, 8, 16) / (den[:, :8].reshape(B, 8, 1) + 1e-16)
    h = h3.reshape(B, 128) + b1_r[...]
    h = jnp.where(h > 0, h, jnp.exp(jnp.minimum(h, 0.0)) - 1.0)
    g = jnp.dot(h, W2_r[...], preferred_element_type=_F32)
    a_s = jnp.sum(g * as2_r[...], axis=-1, keepdims=True)
    a_d = jnp.sum(g * ad2_r[...], axis=-1, keepdims=True)
    hp_r[...] = jnp.concatenate([g, a_s, jnp.zeros((B, 15), _F32)], axis=1)
    adst_r[...] = jnp.concatenate([a_d, jnp.zeros((B, 15), _F32)], axis=1)


def _final_body(p_r, d_r, b2_r, o_r):
    p = p_r[0] + p_r[1]
    den = d_r[0][:, :1] + d_r[1][:, :1]
    o_r[...] = p / (den + 1e-16) + b2_r[...]


def _row_spec(w):
    return pl.BlockSpec((_BLK, w), lambda i: (i, 0))


def _full_spec(shape):
    return pl.BlockSpec(shape, lambda i: tuple(0 for _ in shape))


def kernel(x, edge_index, W1, att_src1, att_dst1, b1, W2, att_src2,
           att_dst2, b2):
    E = edge_index.shape[1]
    e_all = E + _N
    tot = _NW * _K
    cpw = -(-e_all // tot)
    e_pad = cpw * tot

    loop = jnp.arange(_N, dtype=jnp.int32)
    padi = jnp.full((e_pad - e_all,), _N, dtype=jnp.int32)
    src = jnp.concatenate([edge_index[0].astype(jnp.int32), loop, padi])
    dst = jnp.concatenate([edge_index[1].astype(jnp.int32), loop, padi])
    x_pad = jnp.pad(x, ((0, _NPAD - _N), (0, 0)))

    grid = (_NPAD // _BLK,)

    hp1, adst1 = pl.pallas_call(
        _dense1_body,
        grid=grid,
        in_specs=[_row_spec(128), _full_spec((128, 128)),
                  _full_spec((8, 16)), _full_spec((8, 16))],
        out_specs=[_row_spec(144), _row_spec(16)],
        out_shape=[jax.ShapeDtypeStruct((_NPAD, 144), _F32),
                   jax.ShapeDtypeStruct((_NPAD, 16), _F32)],
    )(x_pad, W1, att_src1, att_dst1)

    ep1 = _make_edge_pass((0, 1, 2, 3, 4, 5, 6, 7), cpw, "gat_edges_l1")
    p1, d1 = ep1(hp1, adst1, src, dst)

    hp2, adst2 = pl.pallas_call(
        _dense2_body,
        grid=grid,
        in_specs=[pl.BlockSpec((2, _BLK, 128), lambda i: (0, i, 0)),
                  pl.BlockSpec((2, _BLK, 16), lambda i: (0, i, 0)),
                  _full_spec((1, 128)), _full_spec((128, 128)),
                  _full_spec((1, 128)), _full_spec((1, 128))],
        out_specs=[_row_spec(144), _row_spec(16)],
        out_shape=[jax.ShapeDtypeStruct((_NPAD, 144), _F32),
                   jax.ShapeDtypeStruct((_NPAD, 16), _F32)],
    )(p1, d1, b1.reshape(1, 128), W2, att_src2, att_dst2)

    ep2 = _make_edge_pass((0,) * 8, cpw, "gat_edges_l2")
    p2, d2 = ep2(hp2, adst2, src, dst)

    out = pl.pallas_call(
        _final_body,
        grid=grid,
        in_specs=[pl.BlockSpec((2, _BLK, 128), lambda i: (0, i, 0)),
                  pl.BlockSpec((2, _BLK, 16), lambda i: (0, i, 0)),
                  _full_spec((1, 128))],
        out_specs=_row_spec(128),
        out_shape=jax.ShapeDtypeStruct((_NPAD, 128), _F32),
    )(p2, d2, b2.reshape(1, 128))

    return out[:_N]


# speedup vs baseline: 54.5536x; 1.9350x over previous
import functools

import jax
import jax.numpy as jnp
from jax import lax
from jax.experimental import pallas as pl
from jax.experimental.pallas import tpu as pltpu
from jax.experimental.pallas import tpu_sc as plsc

_N = 10000
_NPAD = 10080
_K = 64
_NW = 32
_BLK = 840
_F32 = jnp.float32


def _splat(vec, lane):
    idx = jnp.full((16, 1), lane, dtype=jnp.int32)
    dnums = lax.GatherDimensionNumbers(
        offset_dims=(), collapsed_slice_dims=(0,), start_index_map=(0,))
    return lax.gather(vec, idx, dnums, (1,),
                      mode=lax.GatherScatterMode.PROMISE_IN_BOUNDS)


def _make_edge_pass(head_of_slice, cpw, name):
    mesh = plsc.VectorSubcoreMesh(core_axis_name="c", subcore_axis_name="s")
    rows_per_sub = _NPAD // 16

    @functools.partial(
        pl.kernel,
        out_type=jax.ShapeDtypeStruct((2, _NPAD, 144), _F32),
        mesh=mesh,
        scratch_types=[
            [pltpu.VMEM((_K,), jnp.int32)] * 2,
            [pltpu.VMEM((_K,), jnp.int32)] * 2,
            [pltpu.VMEM((_K,), jnp.int32)] * 2,
            [pltpu.VMEM((_K, 144), _F32)] * 2,
            [pltpu.VMEM((_K, 16), _F32)] * 2,
            [pltpu.VMEM((_K, 144), _F32)] * 2,
            pltpu.VMEM_SHARED((_NPAD, 144), _F32),
            [pltpu.SemaphoreType.DMA] * 2,
            [pltpu.SemaphoreType.DMA] * 2,
            [pltpu.SemaphoreType.DMA] * 2,
        ],
        compiler_params=pltpu.CompilerParams(use_tc_tiling_on_sc=False),
        name=name,
    )
    def edge_pass(hp, adst, src, dst, outp,
                  src_v, dst_v, sdst_v, hp_b, adst_b, mw_b, acc_s,
                  sem_h, sem_a, sem_s):
        core = lax.axis_index("c")
        sub = lax.axis_index("s")
        wid = sub * 2 + core
        zero16 = jnp.zeros((16,), _F32)

        def issue(slot, ci):
            base = (wid * cpw + ci) * _K
            pltpu.sync_copy(src.at[pl.ds(base, _K)], src_v[slot])
            pltpu.sync_copy(dst.at[pl.ds(base, _K)], dst_v[slot])
            pltpu.async_copy(hp.at[src_v[slot]], hp_b[slot], sem_h[slot])
            pltpu.async_copy(adst.at[dst_v[slot]], adst_b[slot], sem_a[slot])

        def wait_gather(slot):
            pltpu.make_async_copy(hp.at[src_v[slot]], hp_b[slot],
                                  sem_h[slot]).wait()
            pltpu.make_async_copy(adst.at[dst_v[slot]], adst_b[slot],
                                  sem_a[slot]).wait()

        def drain_scatter(slot):
            pltpu.make_async_copy(mw_b[slot], acc_s.at[sdst_v[slot]],
                                  sem_s[slot]).wait()

        def compute(slot):
            @pl.loop(0, _K // 16)
            def _(i):
                sdst_v[slot][pl.ds(i * 16, 16)] = dst_v[slot][pl.ds(i * 16, 16)]

            @pl.loop(0, _K)
            def _(j):
                v = hp_b[slot][j, pl.ds(128, 16)] + adst_b[slot][j]
                w = jnp.exp(jnp.maximum(v, 0.2 * v))
                mw_b[slot][j, pl.ds(128, 16)] = w
                splats = {}
                for c in range(8):
                    ln = head_of_slice[c]
                    if ln not in splats:
                        splats[ln] = _splat(w, ln)
                    mw_b[slot][j, pl.ds(c * 16, 16)] = (
                        hp_b[slot][j, pl.ds(c * 16, 16)] * splats[ln])

        def issue_scatter(slot):
            pltpu.async_copy(mw_b[slot], acc_s.at[sdst_v[slot]], sem_s[slot],
                             add=True)

        @pl.loop(0, _K)
        def _(j):
            for c in range(9):
                mw_b[0][j, pl.ds(c * 16, 16)] = zero16

        @pl.loop(0, rows_per_sub // _K)
        def _(t):
            off = sub * rows_per_sub + t * _K
            pltpu.sync_copy(mw_b[0], acc_s.at[pl.ds(off, _K)])

        rem = rows_per_sub % _K
        if rem:
            off0 = sub * rows_per_sub + (rows_per_sub // _K) * _K
            pltpu.sync_copy(mw_b[0].at[pl.ds(0, rem)],
                            acc_s.at[pl.ds(off0, rem)])

        plsc.subcore_barrier()

        issue(0, 0)

        @pl.loop(0, cpw // 2)
        def _(t):
            c0 = 2 * t
            issue(1, c0 + 1)
            wait_gather(0)

            @pl.when(t > 0)
            def _():
                drain_scatter(0)

            compute(0)
            issue_scatter(0)

            @pl.when(t + 1 < cpw // 2)
            def _():
                issue(0, c0 + 2)

            wait_gather(1)

            @pl.when(t > 0)
            def _():
                drain_scatter(1)

            compute(1)
            issue_scatter(1)

        drain_scatter(0)
        drain_scatter(1)

        plsc.subcore_barrier()
        off = sub * rows_per_sub
        pltpu.sync_copy(acc_s.at[pl.ds(off, rows_per_sub)],
                        outp.at[core, pl.ds(off, rows_per_sub)])

    return edge_pass


def _dense1_body(x_r, W_r, as_r, ad_r, hp_r, adst_r):
    h = jnp.dot(x_r[...], W_r[...], preferred_element_type=_F32)
    B = h.shape[0]
    h3 = h.reshape(B, 8, 16)
    a_s = jnp.sum(h3 * as_r[...][None], axis=-1)
    a_d = jnp.sum(h3 * ad_r[...][None], axis=-1)
    hp_r[...] = jnp.concatenate([h, a_s, jnp.zeros((B, 8), _F32)], axis=1)
    adst_r[...] = jnp.concatenate([a_d, jnp.zeros((B, 8), _F32)], axis=1)


def _dense2_body(p_r, b1_r, W2_r, as2_r, ad2_r, hp_r, adst_r):
    p = p_r[0] + p_r[1]
    B = p.shape[0]
    den = p[:, 128:136]
    h3 = p[:, :128].reshape(B, 8, 16) / (den.reshape(B, 8, 1) + 1e-16)
    h = h3.reshape(B, 128) + b1_r[...]
    h = jnp.where(h > 0, h, jnp.exp(jnp.minimum(h, 0.0)) - 1.0)
    g = jnp.dot(h, W2_r[...], preferred_element_type=_F32)
    a_s = jnp.sum(g * as2_r[...], axis=-1, keepdims=True)
    a_d = jnp.sum(g * ad2_r[...], axis=-1, keepdims=True)
    hp_r[...] = jnp.concatenate([g, a_s, jnp.zeros((B, 15), _F32)], axis=1)
    adst_r[...] = jnp.concatenate([a_d, jnp.zeros((B, 15), _F32)], axis=1)


def _final_body(p_r, b2_r, o_r):
    p = p_r[0] + p_r[1]
    den = p[:, 128:129]
    o_r[...] = p[:, :128] / (den + 1e-16) + b2_r[...]


def _row_spec(w):
    return pl.BlockSpec((_BLK, w), lambda i: (i, 0))


def _full_spec(shape):
    return pl.BlockSpec(shape, lambda i: tuple(0 for _ in shape))


def kernel(x, edge_index, W1, att_src1, att_dst1, b1, W2, att_src2,
           att_dst2, b2):
    E = edge_index.shape[1]
    e_all = E + _N
    tot = 2 * _NW * _K
    cpw = 2 * (-(-e_all // tot))
    e_pad = cpw * _NW * _K

    loop = jnp.arange(_N, dtype=jnp.int32)
    padi = jnp.full((e_pad - e_all,), _N, dtype=jnp.int32)
    src = jnp.concatenate([edge_index[0].astype(jnp.int32), loop, padi])
    dst = jnp.concatenate([edge_index[1].astype(jnp.int32), loop, padi])
    x_pad = jnp.pad(x, ((0, _NPAD - _N), (0, 0)))

    grid = (_NPAD // _BLK,)

    hp1, adst1 = pl.pallas_call(
        _dense1_body,
        grid=grid,
        in_specs=[_row_spec(128), _full_spec((128, 128)),
                  _full_spec((8, 16)), _full_spec((8, 16))],
        out_specs=[_row_spec(144), _row_spec(16)],
        out_shape=[jax.ShapeDtypeStruct((_NPAD, 144), _F32),
                   jax.ShapeDtypeStruct((_NPAD, 16), _F32)],
    )(x_pad, W1, att_src1, att_dst1)

    ep1 = _make_edge_pass((0, 1, 2, 3, 4, 5, 6, 7), cpw, "gat_edges_l1")
    p1 = ep1(hp1, adst1, src, dst)

    hp2, adst2 = pl.pallas_call(
        _dense2_body,
        grid=grid,
        in_specs=[pl.BlockSpec((2, _BLK, 144), lambda i: (0, i, 0)),
                  _full_spec((1, 128)), _full_spec((128, 128)),
                  _full_spec((1, 128)), _full_spec((1, 128))],
        out_specs=[_row_spec(144), _row_spec(16)],
        out_shape=[jax.ShapeDtypeStruct((_NPAD, 144), _F32),
                   jax.ShapeDtypeStruct((_NPAD, 16), _F32)],
    )(p1, b1.reshape(1, 128), W2, att_src2, att_dst2)

    ep2 = _make_edge_pass((0,) * 8, cpw, "gat_edges_l2")
    p2 = ep2(hp2, adst2, src, dst)

    out = pl.pallas_call(
        _final_body,
        grid=grid,
        in_specs=[pl.BlockSpec((2, _BLK, 144), lambda i: (0, i, 0)),
                  _full_spec((1, 128))],
        out_specs=_row_spec(128),
        out_shape=jax.ShapeDtypeStruct((_NPAD, 128), _F32),
    )(p2, b2.reshape(1, 128))

    return out[:_N]


# speedup vs baseline: 69.7331x; 1.2782x over previous
import functools

import jax
import jax.numpy as jnp
from jax import lax
from jax.experimental import pallas as pl
from jax.experimental.pallas import tpu as pltpu
from jax.experimental.pallas import tpu_sc as plsc

_N = 10000
_NPAD = 10080
_K = 64
_NW = 32
_BLK = 840
_F32 = jnp.float32


def _splat(vec, lane):
    idx = jnp.full((16, 1), lane, dtype=jnp.int32)
    dnums = lax.GatherDimensionNumbers(
        offset_dims=(), collapsed_slice_dims=(0,), start_index_map=(0,))
    return lax.gather(vec, idx, dnums, (1,),
                      mode=lax.GatherScatterMode.PROMISE_IN_BOUNDS)


def _make_edge_pass(head_of_slice, cpw, name):
    mesh = plsc.VectorSubcoreMesh(core_axis_name="c", subcore_axis_name="s")
    rows_per_sub = _NPAD // 16

    @functools.partial(
        pl.kernel,
        out_type=jax.ShapeDtypeStruct((2, _NPAD, 144), _F32),
        mesh=mesh,
        scratch_types=[
            [pltpu.VMEM((_K,), jnp.int32)] * 2,
            [pltpu.VMEM((_K,), jnp.int32)] * 2,
            [pltpu.VMEM((_K,), jnp.int32)] * 2,
            [pltpu.VMEM((_K, 144), _F32)] * 2,
            [pltpu.VMEM((_K, 16), _F32)] * 2,
            [pltpu.VMEM((_K, 144), _F32)] * 2,
            pltpu.VMEM_SHARED((_NPAD, 144), _F32),
            [pltpu.SemaphoreType.DMA] * 2,
            [pltpu.SemaphoreType.DMA] * 2,
            [pltpu.SemaphoreType.DMA] * 2,
        ],
        compiler_params=pltpu.CompilerParams(use_tc_tiling_on_sc=False),
        name=name,
    )
    def edge_pass(hp, adst, src, dst, outp,
                  src_v, dst_v, sdst_v, hp_b, adst_b, mw_b, acc_s,
                  sem_h, sem_a, sem_s):
        core = lax.axis_index("c")
        sub = lax.axis_index("s")
        wid = sub * 2 + core
        zero16 = jnp.zeros((16,), _F32)

        def issue(slot, ci):
            base = (wid * cpw + ci) * _K
            pltpu.sync_copy(src.at[pl.ds(base, _K)], src_v[slot])
            pltpu.sync_copy(dst.at[pl.ds(base, _K)], dst_v[slot])
            pltpu.async_copy(hp.at[src_v[slot]], hp_b[slot], sem_h[slot])
            pltpu.async_copy(adst.at[dst_v[slot]], adst_b[slot], sem_a[slot])

        def wait_gather(slot):
            pltpu.make_async_copy(hp.at[src_v[slot]], hp_b[slot],
                                  sem_h[slot]).wait()
            pltpu.make_async_copy(adst.at[dst_v[slot]], adst_b[slot],
                                  sem_a[slot]).wait()

        def drain_scatter(slot):
            pltpu.make_async_copy(mw_b[slot], acc_s.at[sdst_v[slot]],
                                  sem_s[slot]).wait()

        def compute(slot):
            @plsc.parallel_loop(0, _K // 16)
            def _(i):
                sdst_v[slot][pl.ds(i * 16, 16)] = dst_v[slot][pl.ds(i * 16, 16)]

            @plsc.parallel_loop(0, _K, unroll=4)
            def _(j):
                v = hp_b[slot][j, pl.ds(128, 16)] + adst_b[slot][j]
                w = jnp.exp(jnp.maximum(v, 0.2 * v))
                mw_b[slot][j, pl.ds(128, 16)] = w
                splats = {}
                for c in range(8):
                    ln = head_of_slice[c]
                    if ln not in splats:
                        splats[ln] = _splat(w, ln)
                    mw_b[slot][j, pl.ds(c * 16, 16)] = (
                        hp_b[slot][j, pl.ds(c * 16, 16)] * splats[ln])

        def issue_scatter(slot):
            pltpu.async_copy(mw_b[slot], acc_s.at[sdst_v[slot]], sem_s[slot],
                             add=True)

        @pl.loop(0, _K)
        def _(j):
            for c in range(9):
                mw_b[0][j, pl.ds(c * 16, 16)] = zero16

        @pl.loop(0, rows_per_sub // _K)
        def _(t):
            off = sub * rows_per_sub + t * _K
            pltpu.sync_copy(mw_b[0], acc_s.at[pl.ds(off, _K)])

        rem = rows_per_sub % _K
        if rem:
            off0 = sub * rows_per_sub + (rows_per_sub // _K) * _K
            pltpu.sync_copy(mw_b[0].at[pl.ds(0, rem)],
                            acc_s.at[pl.ds(off0, rem)])

        plsc.subcore_barrier()

        issue(0, 0)

        @pl.loop(0, cpw // 2)
        def _(t):
            c0 = 2 * t
            issue(1, c0 + 1)
            wait_gather(0)

            @pl.when(t > 0)
            def _():
                drain_scatter(0)

            compute(0)
            issue_scatter(0)

            @pl.when(t + 1 < cpw // 2)
            def _():
                issue(0, c0 + 2)

            wait_gather(1)

            @pl.when(t > 0)
            def _():
                drain_scatter(1)

            compute(1)
            issue_scatter(1)

        drain_scatter(0)
        drain_scatter(1)

        plsc.subcore_barrier()
        off = sub * rows_per_sub
        pltpu.sync_copy(acc_s.at[pl.ds(off, rows_per_sub)],
                        outp.at[core, pl.ds(off, rows_per_sub)])

    return edge_pass


def _dense1_body(x_r, W_r, as_r, ad_r, hp_r, adst_r):
    h = jnp.dot(x_r[...], W_r[...], preferred_element_type=_F32)
    B = h.shape[0]
    h3 = h.reshape(B, 8, 16)
    a_s = jnp.sum(h3 * as_r[...][None], axis=-1)
    a_d = jnp.sum(h3 * ad_r[...][None], axis=-1)
    hp_r[...] = jnp.concatenate([h, a_s, jnp.zeros((B, 8), _F32)], axis=1)
    adst_r[...] = jnp.concatenate([a_d, jnp.zeros((B, 8), _F32)], axis=1)


def _dense2_body(p_r, b1_r, W2_r, as2_r, ad2_r, hp_r, adst_r):
    p = p_r[0] + p_r[1]
    B = p.shape[0]
    den = p[:, 128:136]
    h3 = p[:, :128].reshape(B, 8, 16) / (den.reshape(B, 8, 1) + 1e-16)
    h = h3.reshape(B, 128) + b1_r[...]
    h = jnp.where(h > 0, h, jnp.exp(jnp.minimum(h, 0.0)) - 1.0)
    g = jnp.dot(h, W2_r[...], preferred_element_type=_F32)
    a_s = jnp.sum(g * as2_r[...], axis=-1, keepdims=True)
    a_d = jnp.sum(g * ad2_r[...], axis=-1, keepdims=True)
    hp_r[...] = jnp.concatenate([g, a_s, jnp.zeros((B, 15), _F32)], axis=1)
    adst_r[...] = jnp.concatenate([a_d, jnp.zeros((B, 15), _F32)], axis=1)


def _final_body(p_r, b2_r, o_r):
    p = p_r[0] + p_r[1]
    den = p[:, 128:129]
    o_r[...] = p[:, :128] / (den + 1e-16) + b2_r[...]


def _row_spec(w):
    return pl.BlockSpec((_BLK, w), lambda i: (i, 0))


def _full_spec(shape):
    return pl.BlockSpec(shape, lambda i: tuple(0 for _ in shape))


def kernel(x, edge_index, W1, att_src1, att_dst1, b1, W2, att_src2,
           att_dst2, b2):
    E = edge_index.shape[1]
    e_all = E + _N
    tot = 2 * _NW * _K
    cpw = 2 * (-(-e_all // tot))
    e_pad = cpw * _NW * _K

    loop = jnp.arange(_N, dtype=jnp.int32)
    padi = jnp.full((e_pad - e_all,), _N, dtype=jnp.int32)
    src = jnp.concatenate([edge_index[0].astype(jnp.int32), loop, padi])
    dst = jnp.concatenate([edge_index[1].astype(jnp.int32), loop, padi])
    x_pad = jnp.pad(x, ((0, _NPAD - _N), (0, 0)))

    grid = (_NPAD // _BLK,)

    hp1, adst1 = pl.pallas_call(
        _dense1_body,
        grid=grid,
        in_specs=[_row_spec(128), _full_spec((128, 128)),
                  _full_spec((8, 16)), _full_spec((8, 16))],
        out_specs=[_row_spec(144), _row_spec(16)],
        out_shape=[jax.ShapeDtypeStruct((_NPAD, 144), _F32),
                   jax.ShapeDtypeStruct((_NPAD, 16), _F32)],
    )(x_pad, W1, att_src1, att_dst1)

    ep1 = _make_edge_pass((0, 1, 2, 3, 4, 5, 6, 7), cpw, "gat_edges_l1")
    p1 = ep1(hp1, adst1, src, dst)

    hp2, adst2 = pl.pallas_call(
        _dense2_body,
        grid=grid,
        in_specs=[pl.BlockSpec((2, _BLK, 144), lambda i: (0, i, 0)),
                  _full_spec((1, 128)), _full_spec((128, 128)),
                  _full_spec((1, 128)), _full_spec((1, 128))],
        out_specs=[_row_spec(144), _row_spec(16)],
        out_shape=[jax.ShapeDtypeStruct((_NPAD, 144), _F32),
                   jax.ShapeDtypeStruct((_NPAD, 16), _F32)],
    )(p1, b1.reshape(1, 128), W2, att_src2, att_dst2)

    ep2 = _make_edge_pass((0,) * 8, cpw, "gat_edges_l2")
    p2 = ep2(hp2, adst2, src, dst)

    out = pl.pallas_call(
        _final_body,
        grid=grid,
        in_specs=[pl.BlockSpec((2, _BLK, 144), lambda i: (0, i, 0)),
                  _full_spec((1, 128))],
        out_specs=_row_spec(128),
        out_shape=jax.ShapeDtypeStruct((_NPAD, 128), _F32),
    )(p2, b2.reshape(1, 128))

    return out[:_N]


# speedup vs baseline: 80.0710x; 1.1482x over previous
import functools

import jax
import jax.numpy as jnp
from jax import lax
from jax.experimental import pallas as pl
from jax.experimental.pallas import tpu as pltpu
from jax.experimental.pallas import tpu_sc as plsc

_N = 10000
_NPAD = 10080
_K = 64
_NW = 32
_BLK = 840
_F32 = jnp.float32


def _splat(vec, lane):
    idx = jnp.full((16, 1), lane, dtype=jnp.int32)
    dnums = lax.GatherDimensionNumbers(
        offset_dims=(), collapsed_slice_dims=(0,), start_index_map=(0,))
    return lax.gather(vec, idx, dnums, (1,),
                      mode=lax.GatherScatterMode.PROMISE_IN_BOUNDS)


def _make_edge_pass(head_of_slice, cpw, name):
    mesh = plsc.VectorSubcoreMesh(core_axis_name="c", subcore_axis_name="s")
    rows_per_sub = _NPAD // 16

    @functools.partial(
        pl.kernel,
        out_type=jax.ShapeDtypeStruct((2, _NPAD, 144), _F32),
        mesh=mesh,
        scratch_types=[
            [pltpu.VMEM((_K,), jnp.int32)] * 2,
            [pltpu.VMEM((_K,), jnp.int32)] * 2,
            [pltpu.VMEM((_K,), jnp.int32)] * 2,
            [pltpu.VMEM((_K, 144), _F32)] * 2,
            [pltpu.VMEM((_K, 16), _F32)] * 2,
            [pltpu.VMEM((_K, 144), _F32)] * 2,
            pltpu.VMEM_SHARED((_NPAD, 144), _F32),
            [pltpu.SemaphoreType.DMA] * 2,
            [pltpu.SemaphoreType.DMA] * 2,
            [pltpu.SemaphoreType.DMA] * 2,
            [pltpu.SemaphoreType.DMA] * 2,
        ],
        compiler_params=pltpu.CompilerParams(use_tc_tiling_on_sc=False),
        name=name,
    )
    def edge_pass(hp, adst, src, dst, outp,
                  src_v, dst_v, sdst_v, hp_b, adst_b, mw_b, acc_s,
                  sem_h, sem_a, sem_s, sem_i):
        core = lax.axis_index("c")
        sub = lax.axis_index("s")
        wid = sub * 2 + core
        zero16 = jnp.zeros((16,), _F32)

        def issue(slot, ci):
            base = (wid * cpw + ci) * _K
            pltpu.async_copy(src.at[pl.ds(base, _K)], src_v[slot], sem_i[slot])
            pltpu.async_copy(dst.at[pl.ds(base, _K)], dst_v[slot], sem_i[slot])
            pltpu.make_async_copy(src.at[pl.ds(base, _K)], src_v[slot],
                                  sem_i[slot]).wait()
            pltpu.make_async_copy(dst.at[pl.ds(base, _K)], dst_v[slot],
                                  sem_i[slot]).wait()
            pltpu.async_copy(hp.at[src_v[slot]], hp_b[slot], sem_h[slot])
            pltpu.async_copy(adst.at[dst_v[slot]], adst_b[slot], sem_a[slot])

        def wait_gather(slot):
            pltpu.make_async_copy(hp.at[src_v[slot]], hp_b[slot],
                                  sem_h[slot]).wait()
            pltpu.make_async_copy(adst.at[dst_v[slot]], adst_b[slot],
                                  sem_a[slot]).wait()

        def drain_scatter(slot):
            pltpu.make_async_copy(mw_b[slot], acc_s.at[sdst_v[slot]],
                                  sem_s[slot]).wait()

        def compute(slot):
            @plsc.parallel_loop(0, _K // 16)
            def _(i):
                sdst_v[slot][pl.ds(i * 16, 16)] = dst_v[slot][pl.ds(i * 16, 16)]

            @plsc.parallel_loop(0, _K, unroll=8)
            def _(j):
                v = hp_b[slot][j, pl.ds(128, 16)] + adst_b[slot][j]
                w = jnp.exp(jnp.maximum(v, 0.2 * v))
                mw_b[slot][j, pl.ds(128, 16)] = w
                splats = {}
                for c in range(8):
                    ln = head_of_slice[c]
                    if ln not in splats:
                        splats[ln] = _splat(w, ln)
                    mw_b[slot][j, pl.ds(c * 16, 16)] = (
                        hp_b[slot][j, pl.ds(c * 16, 16)] * splats[ln])

        def issue_scatter(slot):
            pltpu.async_copy(mw_b[slot], acc_s.at[sdst_v[slot]], sem_s[slot],
                             add=True)

        @pl.loop(0, _K)
        def _(j):
            for c in range(9):
                mw_b[0][j, pl.ds(c * 16, 16)] = zero16

        @pl.loop(0, rows_per_sub // _K)
        def _(t):
            off = sub * rows_per_sub + t * _K
            pltpu.sync_copy(mw_b[0], acc_s.at[pl.ds(off, _K)])

        rem = rows_per_sub % _K
        if rem:
            off0 = sub * rows_per_sub + (rows_per_sub // _K) * _K
            pltpu.sync_copy(mw_b[0].at[pl.ds(0, rem)],
                            acc_s.at[pl.ds(off0, rem)])

        plsc.subcore_barrier()

        issue(0, 0)

        @pl.loop(0, cpw // 2)
        def _(t):
            c0 = 2 * t
            issue(1, c0 + 1)
            wait_gather(0)

            @pl.when(t > 0)
            def _():
                drain_scatter(0)

            compute(0)
            issue_scatter(0)

            @pl.when(t + 1 < cpw // 2)
            def _():
                issue(0, c0 + 2)

            wait_gather(1)

            @pl.when(t > 0)
            def _():
                drain_scatter(1)

            compute(1)
            issue_scatter(1)

        drain_scatter(0)
        drain_scatter(1)

        plsc.subcore_barrier()
        off = sub * rows_per_sub
        pltpu.sync_copy(acc_s.at[pl.ds(off, rows_per_sub)],
                        outp.at[core, pl.ds(off, rows_per_sub)])

    return edge_pass


def _dense1_body(x_r, Wp_r, hp_r, adst_r):
    g = jnp.dot(x_r[...], Wp_r[...], preferred_element_type=_F32)
    hp_r[...] = g[:, :144]
    adst_r[...] = g[:, 144:160]


def _dense2_body(p_r, b1_r, Wp_r, hp_r, adst_r):
    p = p_r[0] + p_r[1]
    B = p.shape[0]
    den = p[:, 128:136]
    h3 = p[:, :128].reshape(B, 8, 16) / (den.reshape(B, 8, 1) + 1e-16)
    h = h3.reshape(B, 128) + b1_r[...]
    h = jnp.where(h > 0, h, jnp.exp(jnp.minimum(h, 0.0)) - 1.0)
    g = jnp.dot(h, Wp_r[...], preferred_element_type=_F32)
    hp_r[...] = g[:, :144]
    adst_r[...] = g[:, 144:160]


def _final_body(p_r, b2_r, o_r):
    p = p_r[0] + p_r[1]
    den = p[:, 128:129]
    o_r[...] = p[:, :128] / (den + 1e-16) + b2_r[...]


def _row_spec(w):
    return pl.BlockSpec((_BLK, w), lambda i: (i, 0))


def _full_spec(shape):
    return pl.BlockSpec(shape, lambda i: tuple(0 for _ in shape))


def kernel(x, edge_index, W1, att_src1, att_dst1, b1, W2, att_src2,
           att_dst2, b2):
    E = edge_index.shape[1]
    e_all = E + _N
    tot = 2 * _NW * _K
    cpw = 2 * (-(-e_all // tot))
    e_pad = cpw * _NW * _K

    loop = jnp.arange(_N, dtype=jnp.int32)
    padi = jnp.full((e_pad - e_all,), _N, dtype=jnp.int32)
    src = jnp.concatenate([edge_index[0].astype(jnp.int32), loop, padi])
    dst = jnp.concatenate([edge_index[1].astype(jnp.int32), loop, padi])
    x_pad = jnp.pad(x, ((0, _NPAD - _N), (0, 0)))

    def pack_w(W, att_s, att_d, heads):
        rep = jnp.repeat(jnp.eye(heads, dtype=_F32), 128 // heads, axis=0)
        As = rep * att_s.reshape(128, 1)
        Ad = rep * att_d.reshape(128, 1)
        z = jnp.zeros((W.shape[0], 16 - heads), _F32)
        return jnp.concatenate([W, W @ As, z, W @ Ad, z], axis=1)

    W1p = pack_w(W1, att_src1, att_dst1, 8)
    W2p = pack_w(W2, att_src2, att_dst2, 1)

    grid = (_NPAD // _BLK,)

    hp1, adst1 = pl.pallas_call(
        _dense1_body,
        grid=grid,
        in_specs=[_row_spec(128), _full_spec((128, 160))],
        out_specs=[_row_spec(144), _row_spec(16)],
        out_shape=[jax.ShapeDtypeStruct((_NPAD, 144), _F32),
                   jax.ShapeDtypeStruct((_NPAD, 16), _F32)],
    )(x_pad, W1p)

    ep1 = _make_edge_pass((0, 1, 2, 3, 4, 5, 6, 7), cpw, "gat_edges_l1")
    p1 = ep1(hp1, adst1, src, dst)

    hp2, adst2 = pl.pallas_call(
        _dense2_body,
        grid=grid,
        in_specs=[pl.BlockSpec((2, _BLK, 144), lambda i: (0, i, 0)),
                  _full_spec((1, 128)), _full_spec((128, 160))],
        out_specs=[_row_spec(144), _row_spec(16)],
        out_shape=[jax.ShapeDtypeStruct((_NPAD, 144), _F32),
                   jax.ShapeDtypeStruct((_NPAD, 16), _F32)],
    )(p1, b1.reshape(1, 128), W2p)

    ep2 = _make_edge_pass((0,) * 8, cpw, "gat_edges_l2")
    p2 = ep2(hp2, adst2, src, dst)

    out = pl.pallas_call(
        _final_body,
        grid=grid,
        in_specs=[pl.BlockSpec((2, _BLK, 144), lambda i: (0, i, 0)),
                  _full_spec((1, 128))],
        out_specs=_row_spec(128),
        out_shape=jax.ShapeDtypeStruct((_NPAD, 128), _F32),
    )(p2, b2.reshape(1, 128))

    return out[:_N]


# speedup vs baseline: 85.5782x; 1.0688x over previous
import functools

import jax
import jax.numpy as jnp
from jax import lax
from jax.experimental import pallas as pl
from jax.experimental.pallas import tpu as pltpu
from jax.experimental.pallas import tpu_sc as plsc

_N = 10000
_NPAD = 10080
_K = 64
_NW = 32
_BLK = 840
_F32 = jnp.float32


def _splat(vec, lane):
    idx = jnp.full((16, 1), lane, dtype=jnp.int32)
    dnums = lax.GatherDimensionNumbers(
        offset_dims=(), collapsed_slice_dims=(0,), start_index_map=(0,))
    return lax.gather(vec, idx, dnums, (1,),
                      mode=lax.GatherScatterMode.PROMISE_IN_BOUNDS)


def _make_edge_pass(head_of_slice, cpw, name):
    mesh = plsc.VectorSubcoreMesh(core_axis_name="c", subcore_axis_name="s")
    rows_per_sub = _NPAD // 16

    @functools.partial(
        pl.kernel,
        out_type=(jax.ShapeDtypeStruct((2, _NPAD, 128), _F32),
                  jax.ShapeDtypeStruct((2, _NPAD, 16), _F32)),
        mesh=mesh,
        scratch_types=[
            [pltpu.VMEM((_K,), jnp.int32)] * 2,
            [pltpu.VMEM((_K,), jnp.int32)] * 2,
            [pltpu.VMEM((_K,), jnp.int32)] * 2,
            [pltpu.VMEM((_K, 128), _F32)] * 2,
            [pltpu.VMEM((_K, 16), _F32)] * 2,
            [pltpu.VMEM((_K, 16), _F32)] * 2,
            [pltpu.VMEM((_K, 144), _F32)] * 2,
            pltpu.VMEM_SHARED((_NPAD, 144), _F32),
            [pltpu.SemaphoreType.DMA] * 2,
            [pltpu.SemaphoreType.DMA] * 2,
            [pltpu.SemaphoreType.DMA] * 2,
            [pltpu.SemaphoreType.DMA] * 2,
            [pltpu.SemaphoreType.DMA] * 2,
        ],
        compiler_params=pltpu.CompilerParams(use_tc_tiling_on_sc=False),
        name=name,
    )
    def edge_pass(h, asrc, adst, src, dst, outp, outd,
                  src_v, dst_v, sdst_v, h_b, asrc_b, adst_b, mw_b, acc_s,
                  sem_h, sem_s1, sem_a, sem_s, sem_i):
        core = lax.axis_index("c")
        sub = lax.axis_index("s")
        wid = sub * 2 + core
        zero16 = jnp.zeros((16,), _F32)

        def issue(slot, ci):
            base = (wid * cpw + ci) * _K
            pltpu.async_copy(src.at[pl.ds(base, _K)], src_v[slot], sem_i[slot])
            pltpu.async_copy(dst.at[pl.ds(base, _K)], dst_v[slot], sem_i[slot])
            pltpu.make_async_copy(src.at[pl.ds(base, _K)], src_v[slot],
                                  sem_i[slot]).wait()
            pltpu.make_async_copy(dst.at[pl.ds(base, _K)], dst_v[slot],
                                  sem_i[slot]).wait()
            pltpu.async_copy(h.at[src_v[slot]], h_b[slot], sem_h[slot])
            pltpu.async_copy(asrc.at[src_v[slot]], asrc_b[slot], sem_s1[slot])
            pltpu.async_copy(adst.at[dst_v[slot]], adst_b[slot], sem_a[slot])

        def wait_gather(slot):
            pltpu.make_async_copy(h.at[src_v[slot]], h_b[slot],
                                  sem_h[slot]).wait()
            pltpu.make_async_copy(asrc.at[src_v[slot]], asrc_b[slot],
                                  sem_s1[slot]).wait()
            pltpu.make_async_copy(adst.at[dst_v[slot]], adst_b[slot],
                                  sem_a[slot]).wait()

        def drain_scatter(slot):
            pltpu.make_async_copy(mw_b[slot], acc_s.at[sdst_v[slot]],
                                  sem_s[slot]).wait()

        def compute(slot):
            @plsc.parallel_loop(0, _K // 16)
            def _(i):
                sdst_v[slot][pl.ds(i * 16, 16)] = dst_v[slot][pl.ds(i * 16, 16)]

            @plsc.parallel_loop(0, _K, unroll=8)
            def _(j):
                v = asrc_b[slot][j] + adst_b[slot][j]
                w = jnp.exp(jnp.maximum(v, 0.2 * v))
                mw_b[slot][j, pl.ds(128, 16)] = w
                splats = {}
                for c in range(8):
                    ln = head_of_slice[c]
                    if ln not in splats:
                        splats[ln] = _splat(w, ln)
                    mw_b[slot][j, pl.ds(c * 16, 16)] = (
                        h_b[slot][j, pl.ds(c * 16, 16)] * splats[ln])

        def issue_scatter(slot):
            pltpu.async_copy(mw_b[slot], acc_s.at[sdst_v[slot]], sem_s[slot],
                             add=True)

        @pl.loop(0, _K)
        def _(j):
            for c in range(9):
                mw_b[0][j, pl.ds(c * 16, 16)] = zero16

        @pl.loop(0, rows_per_sub // _K)
        def _(t):
            off = sub * rows_per_sub + t * _K
            pltpu.sync_copy(mw_b[0], acc_s.at[pl.ds(off, _K)])

        rem = rows_per_sub % _K
        if rem:
            off0 = sub * rows_per_sub + (rows_per_sub // _K) * _K
            pltpu.sync_copy(mw_b[0].at[pl.ds(0, rem)],
                            acc_s.at[pl.ds(off0, rem)])

        plsc.subcore_barrier()

        issue(0, 0)

        @pl.loop(0, cpw // 2)
        def _(t):
            c0 = 2 * t
            issue(1, c0 + 1)
            wait_gather(0)

            @pl.when(t > 0)
            def _():
                drain_scatter(0)

            compute(0)
            issue_scatter(0)

            @pl.when(t + 1 < cpw // 2)
            def _():
                issue(0, c0 + 2)

            wait_gather(1)

            @pl.when(t > 0)
            def _():
                drain_scatter(1)

            compute(1)
            issue_scatter(1)

        drain_scatter(0)
        drain_scatter(1)

        plsc.subcore_barrier()
        off = sub * rows_per_sub
        pltpu.sync_copy(acc_s.at[pl.ds(off, rows_per_sub), pl.ds(0, 128)],
                        outp.at[core, pl.ds(off, rows_per_sub)])
        pltpu.sync_copy(acc_s.at[pl.ds(off, rows_per_sub), pl.ds(128, 16)],
                        outd.at[core, pl.ds(off, rows_per_sub)])

    return edge_pass


def _dense1_body(x_r, Wp_r, h_r, asrc_r, adst_r):
    g = jnp.dot(x_r[...], Wp_r[...], preferred_element_type=_F32)
    h_r[...] = g[:, :128]
    asrc_r[...] = g[:, 128:144]
    adst_r[...] = g[:, 144:160]


def _dense2_body(p_r, d_r, b1_r, Wp_r, h_r, asrc_r, adst_r):
    p = p_r[0] + p_r[1]
    den = d_r[0] + d_r[1]
    B = p.shape[0]
    h3 = p.reshape(B, 8, 16) / (den[:, :8].reshape(B, 8, 1) + 1e-16)
    h = h3.reshape(B, 128) + b1_r[...]
    h = jnp.where(h > 0, h, jnp.exp(jnp.minimum(h, 0.0)) - 1.0)
    g = jnp.dot(h, Wp_r[...], preferred_element_type=_F32)
    h_r[...] = g[:, :128]
    asrc_r[...] = g[:, 128:144]
    adst_r[...] = g[:, 144:160]


def _final_body(p_r, d_r, b2_r, o_r):
    p = p_r[0] + p_r[1]
    den = d_r[0][:, :1] + d_r[1][:, :1]
    o_r[...] = p / (den + 1e-16) + b2_r[...]


def _row_spec(w):
    return pl.BlockSpec((_BLK, w), lambda i: (i, 0))


def _full_spec(shape):
    return pl.BlockSpec(shape, lambda i: tuple(0 for _ in shape))


def kernel(x, edge_index, W1, att_src1, att_dst1, b1, W2, att_src2,
           att_dst2, b2):
    E = edge_index.shape[1]
    e_all = E + _N
    tot = 2 * _NW * _K
    cpw = 2 * (-(-e_all // tot))
    e_pad = cpw * _NW * _K

    loop = jnp.arange(_N, dtype=jnp.int32)
    padi = jnp.full((e_pad - e_all,), _N, dtype=jnp.int32)
    src = jnp.concatenate([edge_index[0].astype(jnp.int32), loop, padi])
    dst = jnp.concatenate([edge_index[1].astype(jnp.int32), loop, padi])
    x_pad = jnp.pad(x, ((0, _NPAD - _N), (0, 0)))

    def pack_w(W, att_s, att_d, heads):
        rep = jnp.repeat(jnp.eye(heads, dtype=_F32), 128 // heads, axis=0)
        As = rep * att_s.reshape(128, 1)
        Ad = rep * att_d.reshape(128, 1)
        z = jnp.zeros((W.shape[0], 16 - heads), _F32)
        return jnp.concatenate([W, W @ As, z, W @ Ad, z], axis=1)

    W1p = pack_w(W1, att_src1, att_dst1, 8)
    W2p = pack_w(W2, att_src2, att_dst2, 1)

    grid = (_NPAD // _BLK,)

    h1, asrc1, adst1 = pl.pallas_call(
        _dense1_body,
        grid=grid,
        in_specs=[_row_spec(128), _full_spec((128, 160))],
        out_specs=[_row_spec(128), _row_spec(16), _row_spec(16)],
        out_shape=[jax.ShapeDtypeStruct((_NPAD, 128), _F32),
                   jax.ShapeDtypeStruct((_NPAD, 16), _F32),
                   jax.ShapeDtypeStruct((_NPAD, 16), _F32)],
    )(x_pad, W1p)

    ep1 = _make_edge_pass((0, 1, 2, 3, 4, 5, 6, 7), cpw, "gat_edges_l1")
    p1, d1 = ep1(h1, asrc1, adst1, src, dst)

    h2, asrc2, adst2 = pl.pallas_call(
        _dense2_body,
        grid=grid,
        in_specs=[pl.BlockSpec((2, _BLK, 128), lambda i: (0, i, 0)),
                  pl.BlockSpec((2, _BLK, 16), lambda i: (0, i, 0)),
                  _full_spec((1, 128)), _full_spec((128, 160))],
        out_specs=[_row_spec(128), _row_spec(16), _row_spec(16)],
        out_shape=[jax.ShapeDtypeStruct((_NPAD, 128), _F32),
                   jax.ShapeDtypeStruct((_NPAD, 16), _F32),
                   jax.ShapeDtypeStruct((_NPAD, 16), _F32)],
    )(p1, d1, b1.reshape(1, 128), W2p)

    ep2 = _make_edge_pass((0,) * 8, cpw, "gat_edges_l2")
    p2, d2 = ep2(h2, asrc2, adst2, src, dst)

    out = pl.pallas_call(
        _final_body,
        grid=grid,
        in_specs=[pl.BlockSpec((2, _BLK, 128), lambda i: (0, i, 0)),
                  pl.BlockSpec((2, _BLK, 16), lambda i: (0, i, 0)),
                  _full_spec((1, 128))],
        out_specs=_row_spec(128),
        out_shape=jax.ShapeDtypeStruct((_NPAD, 128), _F32),
    )(p2, d2, b2.reshape(1, 128))

    return out[:_N]
